# Initial kernel scaffold; baseline (speedup 1.0000x reference)
import jax
import jax.numpy as jnp
from jax.experimental import pallas as pl


def kernel(x, y, fg, iter):
    raise NotImplementedError("write your pallas kernel here")


# speedup vs baseline: 8.4600x; 8.4600x over previous
import functools

import jax
import jax.numpy as jnp
from jax.experimental import pallas as pl

_EPS = 1e-08


def _block_kernel(x_ref, y_ref, fg_ref, out_ref, *, n_total):
    i = pl.program_id(0)
    nb = pl.num_programs(0)
    hw = x_ref.shape[1]
    blk = x_ref.shape[0]

    s = jax.nn.sigmoid(x_ref[...])
    si = jax.lax.bitcast_convert_type(s, jnp.int32)

    fg = fg_ref[0, 0, :]
    y = y_ref[0, 0, :]
    kp1 = (fg * hw).astype(jnp.int32) + 1

    def step(_, carry):
        r, bit = carry
        cand = r | bit
        cnt = jnp.sum((si >= cand[:, None]).astype(jnp.int32), axis=1)
        return jnp.where(cnt >= kp1, cand, r), bit >> 1

    r0 = jnp.zeros((blk,), jnp.int32)
    bit0 = jnp.full((blk,), 1 << 29, jnp.int32)
    r, _ = jax.lax.fori_loop(0, 30, step, (r0, bit0))

    t = jax.lax.bitcast_convert_type(r, jnp.float32)
    t = jnp.maximum(t, 1e-4)[:, None]
    yb = y[:, None]

    u = s * (1.0 / t)
    h_low = u * (2.0 - u)
    one_m_t = 1.0 - t
    alpha = 1.0 / jnp.maximum(one_m_t * one_m_t, _EPS)
    h_high = alpha * (1.0 - s) * (one_m_t + (s - t))
    piece = jnp.where(s <= t, h_low, h_high) * yb
    neg = -(1.0 - yb) * jnp.log(jnp.maximum(1.0 - s, _EPS))
    bsum = jnp.sum(piece + neg)

    prev = jnp.where(i == 0, jnp.zeros((1, 1), jnp.float32), out_ref[...])
    acc = prev + bsum
    out_ref[...] = jnp.where(i == nb - 1, acc / n_total, acc)


@functools.partial(jax.jit, static_argnames=("interpret",))
def _spatial_bce(x, y, fg, interpret=False):
    b, c, h, w = x.shape
    hw = h * w
    rows = b * c
    blk = 256
    nb = rows // blk
    x2 = x.reshape(rows, hw)
    y3 = y.reshape(nb, 1, blk)
    fg3 = fg.reshape(nb, 1, blk)
    out = pl.pallas_call(
        functools.partial(_block_kernel, n_total=rows * hw),
        grid=(nb,),
        in_specs=[
            pl.BlockSpec((blk, hw), lambda i: (i, 0)),
            pl.BlockSpec((1, 1, blk), lambda i: (i, 0, 0)),
            pl.BlockSpec((1, 1, blk), lambda i: (i, 0, 0)),
        ],
        out_specs=pl.BlockSpec((1, 1), lambda i: (0, 0)),
        out_shape=jax.ShapeDtypeStruct((1, 1), jnp.float32),
        interpret=interpret,
    )(x2, y3, fg3)
    return out[0, 0]


def kernel(x, y, fg, iter):
    return _spatial_bce(x, y, fg) + jnp.asarray(iter, jnp.float32) * 0.0


# speedup vs baseline: 9.6851x; 1.1448x over previous
import functools

import jax
import jax.numpy as jnp
from jax.experimental import pallas as pl

_EPS = 1e-08


def _block_kernel(x_ref, y_ref, fg_ref, out_ref, *, n_total):
    i = pl.program_id(0)
    nb = pl.num_programs(0)
    hw = x_ref.shape[1]
    blk = x_ref.shape[0]

    s = jax.nn.sigmoid(x_ref[...])
    si = jax.lax.bitcast_convert_type(s, jnp.int32)

    fg = fg_ref[0, 0, :]
    y = y_ref[0, 0, :]
    kp1 = (fg * hw).astype(jnp.int32) + 1

    def count_true(mask):
        m = mask.astype(jnp.int16)
        n = m.shape[1]
        while n > 128:
            n //= 2
            m = m[:, :n] + m[:, n:]
        return jnp.sum(m.astype(jnp.int32), axis=1)

    def search15(data16, target):
        def step(_, carry):
            r, bit = carry
            cand = r | bit
            cnt = count_true(data16 >= cand.astype(jnp.int16)[:, None])
            return jnp.where(cnt >= target, cand, r), bit >> 1
        r0 = jnp.zeros((blk,), jnp.int32)
        bit0 = jnp.full((blk,), 1 << 14, jnp.int32)
        r, _ = jax.lax.fori_loop(0, 15, step, (r0, bit0))
        return r

    sh16 = (si >> 15).astype(jnp.int16)
    rh = search15(sh16, kp1)

    rh16b = rh.astype(jnp.int16)[:, None]
    eq = sh16 == rh16b
    c_gt = count_true(sh16 > rh16b)
    sl16 = jnp.where(eq, (si & 0x7FFF).astype(jnp.int16), jnp.int16(-1))
    rl = search15(sl16, kp1 - c_gt)

    r = (rh << 15) | rl
    t = jax.lax.bitcast_convert_type(r, jnp.float32)
    t = jnp.maximum(t, 1e-4)[:, None]
    yb = y[:, None]

    u = s * (1.0 / t)
    h_low = u * (2.0 - u)
    one_m_t = 1.0 - t
    alpha = 1.0 / jnp.maximum(one_m_t * one_m_t, _EPS)
    h_high = alpha * (1.0 - s) * (one_m_t + (s - t))
    piece = jnp.where(s <= t, h_low, h_high) * yb
    neg = -(1.0 - yb) * jnp.log(jnp.maximum(1.0 - s, _EPS))
    bsum = jnp.sum(piece + neg)

    prev = jnp.where(i == 0, jnp.zeros((1, 1), jnp.float32), out_ref[...])
    acc = prev + bsum
    out_ref[...] = jnp.where(i == nb - 1, acc / n_total, acc)


@functools.partial(jax.jit, static_argnames=("interpret",))
def _spatial_bce(x, y, fg, interpret=False):
    b, c, h, w = x.shape
    hw = h * w
    rows = b * c
    blk = 256
    nb = rows // blk
    x2 = x.reshape(rows, hw)
    y3 = y.reshape(nb, 1, blk)
    fg3 = fg.reshape(nb, 1, blk)
    out = pl.pallas_call(
        functools.partial(_block_kernel, n_total=rows * hw),
        grid=(nb,),
        in_specs=[
            pl.BlockSpec((blk, hw), lambda i: (i, 0)),
            pl.BlockSpec((1, 1, blk), lambda i: (i, 0, 0)),
            pl.BlockSpec((1, 1, blk), lambda i: (i, 0, 0)),
        ],
        out_specs=pl.BlockSpec((1, 1), lambda i: (0, 0)),
        out_shape=jax.ShapeDtypeStruct((1, 1), jnp.float32),
        interpret=interpret,
    )(x2, y3, fg3)
    return out[0, 0]


def kernel(x, y, fg, iter):
    return _spatial_bce(x, y, fg) + jnp.asarray(iter, jnp.float32) * 0.0
